# Initial kernel scaffold; baseline (speedup 1.0000x reference)
import functools

import jax
import jax.numpy as jnp
from jax import lax
from jax.experimental import pallas as pl
from jax.experimental.pallas import tpu as pltpu
from jax.experimental.pallas import tpu_sc as plsc

F32 = jnp.float32

N = 10000
E = 320000
DI = 128
H2 = 128
HID = 256

NSUB = 16
NCORE = 2
BE = 128
RPT = 640
NP = NSUB * RPT

DW = DI + 16

EG = E + N
NBG = -(-EG // (NSUB * BE))
NBS = -(-E // (NSUB * BE))
EGP = NSUB * NBG * BE
ESP = NSUB * NBS * BE

RB = 512
GRID = NP // RB


def _dot(a, b):
    return jnp.dot(a, b, preferred_element_type=F32)


def _tc_pre_body(x_ref, wg_ref, asrc_ref, adst_ref,
                 xwp_ref, xp_ref, as_ref, ad_ref):
    x = x_ref[...]
    xw = _dot(x, wg_ref[...])
    ones = jnp.ones((RB, 1), F32)
    zeros = jnp.zeros((RB, DW - DI - 1), F32)
    xwp_ref[...] = jnp.concatenate([xw, ones, zeros], axis=1)
    xp_ref[...] = jnp.concatenate([x, ones, zeros], axis=1)
    as_ref[...] = _dot(xw, asrc_ref[...])
    ad_ref[...] = _dot(xw, adst_ref[...])


def _tc_pre(xP, W_gat, a_src, a_dst):
    row = lambda i: (i, 0)
    full = lambda i: (0, 0)
    return pl.pallas_call(
        _tc_pre_body,
        grid=(GRID,),
        in_specs=[
            pl.BlockSpec((RB, DI), row),
            pl.BlockSpec((DI, H2), full),
            pl.BlockSpec((H2, 1), full),
            pl.BlockSpec((H2, 1), full),
        ],
        out_specs=[
            pl.BlockSpec((RB, DW), row),
            pl.BlockSpec((RB, DW), row),
            pl.BlockSpec((RB, 1), row),
            pl.BlockSpec((RB, 1), row),
        ],
        out_shape=[
            jax.ShapeDtypeStruct((NP, DW), F32),
            jax.ShapeDtypeStruct((NP, DW), F32),
            jax.ShapeDtypeStruct((NP, 1), F32),
            jax.ShapeDtypeStruct((NP, 1), F32),
        ],
    )(xP, W_gat, a_src, a_dst)


def _layernorm(h, g, b):
    mu = jnp.mean(h, axis=1, keepdims=True)
    d = h - mu
    var = jnp.mean(d * d, axis=1, keepdims=True)
    return d * lax.rsqrt(var + 1e-5) * g + b


def _tc_mid_body(md_ref, ag_ref, x_ref, bgat_ref, w1l_ref, b1l_ref,
                 w1r_ref, g1_ref, be1_ref, w2r_ref,
                 hlo_ref, hhi_ref, hr_ref):
    md = md_ref[...]
    ag = ag_ref[...]
    x1 = md[:, :H2] / jnp.maximum(md[:, H2:H2 + 1], 1e-16) + bgat_ref[...]
    mean1 = ag[:, :DI] / jnp.maximum(ag[:, DI:DI + 1], 1.0)
    x2 = _dot(mean1, w1l_ref[...]) + b1l_ref[...] + _dot(x_ref[...], w1r_ref[...])
    h = jnp.concatenate([x1, x2], axis=1)
    h = _layernorm(h, g1_ref[...], be1_ref[...])
    h = jnp.maximum(h, 0.0)
    hlo_ref[...] = h[:, :H2]
    hhi_ref[...] = h[:, H2:]
    hr_ref[...] = _dot(h, w2r_ref[...])


def _tc_mid(md, ag, xP, bgat, W1_l, b1l, W1_r, g1, be1, W2_r):
    row = lambda i: (i, 0)
    full = lambda i: (0, 0)
    return pl.pallas_call(
        _tc_mid_body,
        grid=(GRID,),
        in_specs=[
            pl.BlockSpec((RB, DW), row),
            pl.BlockSpec((RB, DW), row),
            pl.BlockSpec((RB, DI), row),
            pl.BlockSpec((1, H2), full),
            pl.BlockSpec((DI, H2), full),
            pl.BlockSpec((1, H2), full),
            pl.BlockSpec((DI, H2), full),
            pl.BlockSpec((1, HID), full),
            pl.BlockSpec((1, HID), full),
            pl.BlockSpec((HID, HID), full),
        ],
        out_specs=[
            pl.BlockSpec((RB, H2), row),
            pl.BlockSpec((RB, H2), row),
            pl.BlockSpec((RB, HID), row),
        ],
        out_shape=[
            jax.ShapeDtypeStruct((NP, H2), F32),
            jax.ShapeDtypeStruct((NP, H2), F32),
            jax.ShapeDtypeStruct((NP, HID), F32),
        ],
    )(md, ag, xP, bgat, W1_l, b1l, W1_r, g1, be1, W2_r)


def _tc_fin_body(alo_ref, ahi_ref, deg_ref, hr_ref, w2l_ref, b2l_ref,
                 g2_ref, be2_ref, wc_ref, bc_ref, out_ref):
    deg = jnp.maximum(deg_ref[...], 1.0)
    m2l = alo_ref[...] / deg
    m2h = ahi_ref[...] / deg
    w2l = w2l_ref[...]
    h2 = (_dot(m2l, w2l[:H2, :]) + _dot(m2h, w2l[H2:, :])
          + b2l_ref[...] + hr_ref[...])
    h2 = _layernorm(h2, g2_ref[...], be2_ref[...])
    h2 = jnp.maximum(h2, 0.0)
    out_ref[...] = _dot(h2, wc_ref[...]) + bc_ref[...]


def _tc_fin(alo, ahi, deg, hr, W2_l, b2l, g2, be2, Wc, bc):
    row = lambda i: (i, 0)
    full = lambda i: (0, 0)
    return pl.pallas_call(
        _tc_fin_body,
        grid=(GRID,),
        in_specs=[
            pl.BlockSpec((RB, H2), row),
            pl.BlockSpec((RB, H2), row),
            pl.BlockSpec((RB, 1), row),
            pl.BlockSpec((RB, HID), row),
            pl.BlockSpec((HID, HID), full),
            pl.BlockSpec((1, HID), full),
            pl.BlockSpec((1, HID), full),
            pl.BlockSpec((1, HID), full),
            pl.BlockSpec((HID, 1), full),
            pl.BlockSpec((1, 1), full),
        ],
        out_specs=pl.BlockSpec((RB, 1), row),
        out_shape=jax.ShapeDtypeStruct((NP, 1), F32),
    )(alo, ahi, deg, hr, W2_l, b2l, g2, be2, Wc, bc)


def _zero_acc(zbuf, acc, width, sid):
    zv = jnp.zeros((16,), F32)
    for r in range(16):
        for k in range(width // 16):
            zbuf[r, pl.ds(k * 16, 16)] = zv
    base = sid * RPT

    def body(i, _):
        off = pl.multiple_of(base + i * 16, 16)
        pltpu.sync_copy(zbuf, acc.at[pl.ds(off, 16)])
        return 0

    lax.fori_loop(0, RPT // 16, body, 0)


def _edge_loop_plain(nblk, sid, src2, dst2, table, sidx, didx, rows, sem, acc):
    def body(b, _):
        bid = sid * nblk + b
        pltpu.sync_copy(src2.at[bid], sidx)
        pltpu.sync_copy(dst2.at[bid], didx)
        pltpu.async_copy(table.at[sidx], rows, sem).wait()
        pltpu.sync_copy(rows, acc.at[didx], add=True)
        return 0

    lax.fori_loop(0, nblk, body, 0)


def _sc_stage_a_body(xwp, xp, as_t, ad_t, gsrc2, gdst2, ssrc2, sdst2,
                     md_out, ag_out,
                     asb, adb, sidx, didx, rows, wbuf, zbuf, sem, acc):
    cid = lax.axis_index("c")
    sid = lax.axis_index("s")

    _zero_acc(zbuf, acc, DW, sid)
    plsc.subcore_barrier()

    @pl.when(cid == 0)
    def _():
        pltpu.sync_copy(as_t, asb)
        pltpu.sync_copy(ad_t, adb)

        def body(b, _):
            bid = sid * NBG + b
            pltpu.sync_copy(gsrc2.at[bid], sidx)
            pltpu.sync_copy(gdst2.at[bid], didx)
            pltpu.async_copy(xwp.at[sidx], rows, sem).wait()
            for j8 in range(BE // 16):
                sl = pl.ds(j8 * 16, 16)
                si = sidx[sl]
                di = didx[sl]
                u = plsc.load_gather(asb, [si]) + plsc.load_gather(adb, [di])
                wbuf[sl] = jnp.exp(jnp.maximum(u, 0.2 * u))
            for j in range(BE):
                w = wbuf[j]
                for k in range(DW // 16):
                    sl = pl.ds(k * 16, 16)
                    rows[j, sl] = rows[j, sl] * w
            pltpu.sync_copy(rows, acc.at[didx], add=True)
            return 0

        lax.fori_loop(0, NBG, body, 0)

    @pl.when(cid == 1)
    def _():
        _edge_loop_plain(NBS, sid, ssrc2, sdst2, xp, sidx, didx, rows, sem, acc)

    plsc.subcore_barrier()

    base = pl.multiple_of(sid * RPT, 16)

    @pl.when(cid == 0)
    def _():
        pltpu.sync_copy(acc.at[pl.ds(base, RPT)], md_out.at[pl.ds(base, RPT)])

    @pl.when(cid == 1)
    def _():
        pltpu.sync_copy(acc.at[pl.ds(base, RPT)], ag_out.at[pl.ds(base, RPT)])


def _sc_stage_a(xwp, xp, as_t, ad_t, gsrc2, gdst2, ssrc2, sdst2):
    mesh = plsc.VectorSubcoreMesh(core_axis_name="c", subcore_axis_name="s")
    return pl.kernel(
        _sc_stage_a_body,
        out_type=[
            jax.ShapeDtypeStruct((NP, DW), F32),
            jax.ShapeDtypeStruct((NP, DW), F32),
        ],
        mesh=mesh,
        scratch_types=[
            pltpu.VMEM((NP,), F32),
            pltpu.VMEM((NP,), F32),
            pltpu.VMEM((BE,), jnp.int32),
            pltpu.VMEM((BE,), jnp.int32),
            pltpu.VMEM((BE, DW), F32),
            pltpu.VMEM((BE,), F32),
            pltpu.VMEM((16, DW), F32),
            pltpu.SemaphoreType.DMA,
            pltpu.VMEM_SHARED((NP, DW), F32),
        ],
    )(xwp, xp, as_t, ad_t, gsrc2, gdst2, ssrc2, sdst2)


def _sc_stage_c_body(hlo, hhi, ssrc2, sdst2, alo_out, ahi_out,
                     sidx, didx, rows, zbuf, sem, acc):
    cid = lax.axis_index("c")
    sid = lax.axis_index("s")

    _zero_acc(zbuf, acc, H2, sid)
    plsc.subcore_barrier()

    @pl.when(cid == 0)
    def _():
        _edge_loop_plain(NBS, sid, ssrc2, sdst2, hlo, sidx, didx, rows, sem, acc)

    @pl.when(cid == 1)
    def _():
        _edge_loop_plain(NBS, sid, ssrc2, sdst2, hhi, sidx, didx, rows, sem, acc)

    plsc.subcore_barrier()

    base = pl.multiple_of(sid * RPT, 16)

    @pl.when(cid == 0)
    def _():
        pltpu.sync_copy(acc.at[pl.ds(base, RPT)], alo_out.at[pl.ds(base, RPT)])

    @pl.when(cid == 1)
    def _():
        pltpu.sync_copy(acc.at[pl.ds(base, RPT)], ahi_out.at[pl.ds(base, RPT)])


def _sc_stage_c(hlo, hhi, ssrc2, sdst2):
    mesh = plsc.VectorSubcoreMesh(core_axis_name="c", subcore_axis_name="s")
    return pl.kernel(
        _sc_stage_c_body,
        out_type=[
            jax.ShapeDtypeStruct((NP, H2), F32),
            jax.ShapeDtypeStruct((NP, H2), F32),
        ],
        mesh=mesh,
        scratch_types=[
            pltpu.VMEM((BE,), jnp.int32),
            pltpu.VMEM((BE,), jnp.int32),
            pltpu.VMEM((BE, H2), F32),
            pltpu.VMEM((16, H2), F32),
            pltpu.SemaphoreType.DMA,
            pltpu.VMEM_SHARED((NP, H2), F32),
        ],
    )(hlo, hhi, ssrc2, sdst2)


def kernel(x, edge_index, W_gat, att_src, att_dst, b_gat, W1_l, b1_l, W1_r,
           g1, be1, W2_l, b2_l, W2_r, g2, be2, Wc, bc):
    src = edge_index[0]
    dst = edge_index[1]
    loops = jnp.arange(N, dtype=jnp.int32)

    gsrc = jnp.concatenate([src, loops])
    gdst = jnp.concatenate([dst, loops])
    gsrc2 = jnp.pad(gsrc, (0, EGP - EG)).reshape(NSUB * NBG, BE)
    gdst2 = jnp.pad(gdst, (0, EGP - EG), constant_values=N).reshape(NSUB * NBG, BE)
    ssrc2 = jnp.pad(src, (0, ESP - E)).reshape(NSUB * NBS, BE)
    sdst2 = jnp.pad(dst, (0, ESP - E), constant_values=N).reshape(NSUB * NBS, BE)

    xP = jnp.pad(x, ((0, NP - N), (0, 0)))

    xwp, xp, as2, ad2 = _tc_pre(xP, W_gat,
                                att_src.reshape(H2, 1), att_dst.reshape(H2, 1))

    md, ag = _sc_stage_a(xwp, xp, as2.reshape(NP), ad2.reshape(NP),
                         gsrc2, gdst2, ssrc2, sdst2)

    hlo, hhi, hr = _tc_mid(md, ag, xP, b_gat.reshape(1, H2), W1_l,
                           b1_l.reshape(1, H2), W1_r, g1.reshape(1, HID),
                           be1.reshape(1, HID), W2_r)

    alo, ahi = _sc_stage_c(hlo, hhi, ssrc2, sdst2)

    deg = lax.slice(ag, (0, DI), (NP, DI + 1))
    out = _tc_fin(alo, ahi, deg, hr, W2_l, b2_l.reshape(1, HID),
                  g2.reshape(1, HID), be2.reshape(1, HID), Wc,
                  bc.reshape(1, 1))
    return out[:N, 0]


# speedup vs baseline: 8.8083x; 8.8083x over previous
import functools

import jax
import jax.numpy as jnp
from jax import lax
from jax.experimental import pallas as pl
from jax.experimental.pallas import tpu as pltpu
from jax.experimental.pallas import tpu_sc as plsc

F32 = jnp.float32

N = 10000
E = 320000
DI = 128
H2 = 128
HID = 256

NSUB = 16
NCORE = 2
BE = 128
RPT = 640
NP = NSUB * RPT

DW = DI + 16

EG = E + N
NBG = -(-EG // (NSUB * BE))
NBS = -(-E // (NSUB * BE))
EGP = NSUB * NBG * BE
ESP = NSUB * NBS * BE

RB = 512
GRID = NP // RB


def _dot(a, b):
    return jnp.dot(a, b, preferred_element_type=F32)


def _tc_pre_body(x_ref, wg_ref, asrc_ref, adst_ref,
                 xwp_ref, xp_ref, adt_ref):
    x = x_ref[...]
    xw = _dot(x, wg_ref[...])
    as_ = _dot(xw, asrc_ref[...])
    ad_ = _dot(xw, adst_ref[...])
    ones = jnp.ones((RB, 1), F32)
    xwp_ref[...] = jnp.concatenate(
        [xw, ones, as_, jnp.zeros((RB, DW - DI - 2), F32)], axis=1)
    xp_ref[...] = jnp.concatenate(
        [x, ones, jnp.zeros((RB, DW - DI - 1), F32)], axis=1)
    adt_ref[...] = jnp.concatenate([ad_, jnp.zeros((RB, 15), F32)], axis=1)


def _tc_pre(xP, W_gat, a_src, a_dst):
    row = lambda i: (i, 0)
    full = lambda i: (0, 0)
    return pl.pallas_call(
        _tc_pre_body,
        grid=(GRID,),
        in_specs=[
            pl.BlockSpec((RB, DI), row),
            pl.BlockSpec((DI, H2), full),
            pl.BlockSpec((H2, 1), full),
            pl.BlockSpec((H2, 1), full),
        ],
        out_specs=[
            pl.BlockSpec((RB, DW), row),
            pl.BlockSpec((RB, DW), row),
            pl.BlockSpec((RB, 16), row),
        ],
        out_shape=[
            jax.ShapeDtypeStruct((NP, DW), F32),
            jax.ShapeDtypeStruct((NP, DW), F32),
            jax.ShapeDtypeStruct((NP, 16), F32),
        ],
    )(xP, W_gat, a_src, a_dst)


def _layernorm(h, g, b):
    mu = jnp.mean(h, axis=1, keepdims=True)
    d = h - mu
    var = jnp.mean(d * d, axis=1, keepdims=True)
    return d * lax.rsqrt(var + 1e-5) * g + b


def _tc_mid_body(md_ref, ag_ref, x_ref, bgat_ref, w1l_ref, b1l_ref,
                 w1r_ref, g1_ref, be1_ref, w2r_ref,
                 hlo_ref, hhi_ref, hr_ref):
    md = md_ref[...]
    ag = ag_ref[...]
    x1 = md[:, :H2] / jnp.maximum(md[:, H2:H2 + 1], 1e-16) + bgat_ref[...]
    mean1 = ag[:, :DI] / jnp.maximum(ag[:, DI:DI + 1], 1.0)
    x2 = _dot(mean1, w1l_ref[...]) + b1l_ref[...] + _dot(x_ref[...], w1r_ref[...])
    h = jnp.concatenate([x1, x2], axis=1)
    h = _layernorm(h, g1_ref[...], be1_ref[...])
    h = jnp.maximum(h, 0.0)
    hlo_ref[...] = h[:, :H2]
    hhi_ref[...] = h[:, H2:]
    hr_ref[...] = _dot(h, w2r_ref[...])


def _tc_mid(md, ag, xP, bgat, W1_l, b1l, W1_r, g1, be1, W2_r):
    row = lambda i: (i, 0)
    full = lambda i: (0, 0)
    return pl.pallas_call(
        _tc_mid_body,
        grid=(GRID,),
        in_specs=[
            pl.BlockSpec((RB, DW), row),
            pl.BlockSpec((RB, DW), row),
            pl.BlockSpec((RB, DI), row),
            pl.BlockSpec((1, H2), full),
            pl.BlockSpec((DI, H2), full),
            pl.BlockSpec((1, H2), full),
            pl.BlockSpec((DI, H2), full),
            pl.BlockSpec((1, HID), full),
            pl.BlockSpec((1, HID), full),
            pl.BlockSpec((HID, HID), full),
        ],
        out_specs=[
            pl.BlockSpec((RB, H2), row),
            pl.BlockSpec((RB, H2), row),
            pl.BlockSpec((RB, HID), row),
        ],
        out_shape=[
            jax.ShapeDtypeStruct((NP, H2), F32),
            jax.ShapeDtypeStruct((NP, H2), F32),
            jax.ShapeDtypeStruct((NP, HID), F32),
        ],
    )(md, ag, xP, bgat, W1_l, b1l, W1_r, g1, be1, W2_r)


def _tc_fin_body(alo_ref, ahi_ref, deg_ref, hr_ref, w2l_ref, b2l_ref,
                 g2_ref, be2_ref, wc_ref, bc_ref, out_ref):
    deg = jnp.maximum(deg_ref[...], 1.0)
    m2l = alo_ref[...] / deg
    m2h = ahi_ref[...] / deg
    w2l = w2l_ref[...]
    h2 = (_dot(m2l, w2l[:H2, :]) + _dot(m2h, w2l[H2:, :])
          + b2l_ref[...] + hr_ref[...])
    h2 = _layernorm(h2, g2_ref[...], be2_ref[...])
    h2 = jnp.maximum(h2, 0.0)
    out_ref[...] = _dot(h2, wc_ref[...]) + bc_ref[...]


def _tc_fin(alo, ahi, deg, hr, W2_l, b2l, g2, be2, Wc, bc):
    row = lambda i: (i, 0)
    full = lambda i: (0, 0)
    return pl.pallas_call(
        _tc_fin_body,
        grid=(GRID,),
        in_specs=[
            pl.BlockSpec((RB, H2), row),
            pl.BlockSpec((RB, H2), row),
            pl.BlockSpec((RB, 1), row),
            pl.BlockSpec((RB, HID), row),
            pl.BlockSpec((HID, HID), full),
            pl.BlockSpec((1, HID), full),
            pl.BlockSpec((1, HID), full),
            pl.BlockSpec((1, HID), full),
            pl.BlockSpec((HID, 1), full),
            pl.BlockSpec((1, 1), full),
        ],
        out_specs=pl.BlockSpec((RB, 1), row),
        out_shape=jax.ShapeDtypeStruct((NP, 1), F32),
    )(alo, ahi, deg, hr, W2_l, b2l, g2, be2, Wc, bc)


def _zero_acc(zbuf, acc, width, sid):
    zv = jnp.zeros((16,), F32)
    for r in range(16):
        for k in range(width // 16):
            zbuf[r, pl.ds(k * 16, 16)] = zv
    base = sid * RPT

    def body(i, _):
        off = pl.multiple_of(base + i * 16, 16)
        pltpu.sync_copy(zbuf, acc.at[pl.ds(off, 16)])
        return 0

    lax.fori_loop(0, RPT // 16, body, 0)


def _edge_loop_plain(nblk, sid, src2, dst2, table, sidx, didx, rows, sem, acc):
    def body(b, _):
        bid = sid * nblk + b
        pltpu.sync_copy(src2.at[bid], sidx)
        pltpu.sync_copy(dst2.at[bid], didx)
        pltpu.async_copy(table.at[sidx], rows, sem).wait()
        pltpu.sync_copy(rows, acc.at[didx], add=True)
        return 0

    lax.fori_loop(0, nblk, body, 0)


def _sc_stage_a_body(xwp, xp, adt, gsrc2, gdst2, ssrc2, sdst2,
                     md_out, ag_out,
                     sidx, didx, rows, adrows, zbuf, sem, acc):
    cid = lax.axis_index("c")
    sid = lax.axis_index("s")

    _zero_acc(zbuf, acc, DW, sid)
    plsc.subcore_barrier()

    @pl.when(cid == 0)
    def _():
        lane = lax.iota(jnp.int32, 16)
        as_col = jnp.full((16,), DI + 1, jnp.int32)
        zero_col = jnp.zeros((16,), jnp.int32)

        def body(b, _):
            bid = sid * NBG + b
            pltpu.sync_copy(gsrc2.at[bid], sidx)
            pltpu.sync_copy(gdst2.at[bid], didx)
            pltpu.async_copy(xwp.at[sidx], rows, sem).wait()
            pltpu.async_copy(adt.at[didx], adrows, sem).wait()
            for j8 in range(BE // 16):
                jvec = lane + (j8 * 16)
                u = (plsc.load_gather(rows, [jvec, as_col])
                     + plsc.load_gather(adrows, [jvec, zero_col]))
                wv = jnp.exp(jnp.maximum(u, 0.2 * u))
                for l in range(16):
                    w = wv[l]
                    j = j8 * 16 + l
                    for k in range(DW // 16):
                        ksl = pl.ds(k * 16, 16)
                        rows[j, ksl] = rows[j, ksl] * w
            pltpu.sync_copy(rows, acc.at[didx], add=True)
            return 0

        lax.fori_loop(0, NBG, body, 0)

    @pl.when(cid == 1)
    def _():
        _edge_loop_plain(NBS, sid, ssrc2, sdst2, xp, sidx, didx, rows, sem, acc)

    plsc.subcore_barrier()

    base = pl.multiple_of(sid * RPT, 16)

    @pl.when(cid == 0)
    def _():
        pltpu.sync_copy(acc.at[pl.ds(base, RPT)], md_out.at[pl.ds(base, RPT)])

    @pl.when(cid == 1)
    def _():
        pltpu.sync_copy(acc.at[pl.ds(base, RPT)], ag_out.at[pl.ds(base, RPT)])


def _sc_stage_a(xwp, xp, adt, gsrc2, gdst2, ssrc2, sdst2):
    mesh = plsc.VectorSubcoreMesh(core_axis_name="c", subcore_axis_name="s")
    return pl.kernel(
        _sc_stage_a_body,
        out_type=[
            jax.ShapeDtypeStruct((NP, DW), F32),
            jax.ShapeDtypeStruct((NP, DW), F32),
        ],
        mesh=mesh,
        compiler_params=pltpu.CompilerParams(needs_layout_passes=False, use_tc_tiling_on_sc=False),
        scratch_types=[
            pltpu.VMEM((BE,), jnp.int32),
            pltpu.VMEM((BE,), jnp.int32),
            pltpu.VMEM((BE, DW), F32),
            pltpu.VMEM((BE, 16), F32),
            pltpu.VMEM((16, DW), F32),
            pltpu.SemaphoreType.DMA,
            pltpu.VMEM_SHARED((NP, DW), F32),
        ],
    )(xwp, xp, adt, gsrc2, gdst2, ssrc2, sdst2)


def _sc_stage_c_body(hlo, hhi, ssrc2, sdst2, alo_out, ahi_out,
                     sidx, didx, rows, zbuf, sem, acc):
    cid = lax.axis_index("c")
    sid = lax.axis_index("s")

    _zero_acc(zbuf, acc, H2, sid)
    plsc.subcore_barrier()

    @pl.when(cid == 0)
    def _():
        _edge_loop_plain(NBS, sid, ssrc2, sdst2, hlo, sidx, didx, rows, sem, acc)

    @pl.when(cid == 1)
    def _():
        _edge_loop_plain(NBS, sid, ssrc2, sdst2, hhi, sidx, didx, rows, sem, acc)

    plsc.subcore_barrier()

    base = pl.multiple_of(sid * RPT, 16)

    @pl.when(cid == 0)
    def _():
        pltpu.sync_copy(acc.at[pl.ds(base, RPT)], alo_out.at[pl.ds(base, RPT)])

    @pl.when(cid == 1)
    def _():
        pltpu.sync_copy(acc.at[pl.ds(base, RPT)], ahi_out.at[pl.ds(base, RPT)])


def _sc_stage_c(hlo, hhi, ssrc2, sdst2):
    mesh = plsc.VectorSubcoreMesh(core_axis_name="c", subcore_axis_name="s")
    return pl.kernel(
        _sc_stage_c_body,
        out_type=[
            jax.ShapeDtypeStruct((NP, H2), F32),
            jax.ShapeDtypeStruct((NP, H2), F32),
        ],
        mesh=mesh,
        compiler_params=pltpu.CompilerParams(needs_layout_passes=False, use_tc_tiling_on_sc=False),
        scratch_types=[
            pltpu.VMEM((BE,), jnp.int32),
            pltpu.VMEM((BE,), jnp.int32),
            pltpu.VMEM((BE, H2), F32),
            pltpu.VMEM((16, H2), F32),
            pltpu.SemaphoreType.DMA,
            pltpu.VMEM_SHARED((NP, H2), F32),
        ],
    )(hlo, hhi, ssrc2, sdst2)


def kernel(x, edge_index, W_gat, att_src, att_dst, b_gat, W1_l, b1_l, W1_r,
           g1, be1, W2_l, b2_l, W2_r, g2, be2, Wc, bc):
    src = edge_index[0]
    dst = edge_index[1]
    loops = jnp.arange(N, dtype=jnp.int32)

    gsrc = jnp.concatenate([src, loops])
    gdst = jnp.concatenate([dst, loops])
    gsrc2 = jnp.pad(gsrc, (0, EGP - EG)).reshape(NSUB * NBG, BE)
    gdst2 = jnp.pad(gdst, (0, EGP - EG), constant_values=N).reshape(NSUB * NBG, BE)
    ssrc2 = jnp.pad(src, (0, ESP - E)).reshape(NSUB * NBS, BE)
    sdst2 = jnp.pad(dst, (0, ESP - E), constant_values=N).reshape(NSUB * NBS, BE)

    xP = jnp.pad(x, ((0, NP - N), (0, 0)))

    xwp, xp, adt = _tc_pre(xP, W_gat,
                           att_src.reshape(H2, 1), att_dst.reshape(H2, 1))

    md, ag = _sc_stage_a(xwp, xp, adt, gsrc2, gdst2, ssrc2, sdst2)

    hlo, hhi, hr = _tc_mid(md, ag, xP, b_gat.reshape(1, H2), W1_l,
                           b1_l.reshape(1, H2), W1_r, g1.reshape(1, HID),
                           be1.reshape(1, HID), W2_r)

    alo, ahi = _sc_stage_c(hlo, hhi, ssrc2, sdst2)

    deg = lax.slice(ag, (0, DI), (NP, DI + 1))
    out = _tc_fin(alo, ahi, deg, hr, W2_l, b2_l.reshape(1, HID),
                  g2.reshape(1, HID), be2.reshape(1, HID), Wc,
                  bc.reshape(1, 1))
    return out[:N, 0]


# speedup vs baseline: 11.3052x; 1.2835x over previous
import functools

import jax
import jax.numpy as jnp
from jax import lax
from jax.experimental import pallas as pl
from jax.experimental.pallas import tpu as pltpu
from jax.experimental.pallas import tpu_sc as plsc

F32 = jnp.float32

N = 10000
E = 320000
DI = 128
H2 = 128
HID = 256

NSUB = 16
NCORE = 2
BG = 96
BC = 128
RPT = 640
NP = NSUB * RPT

DW = DI + 16


def _even_blocks(edges, be):
    nb = -(-edges // (NSUB * be))
    return nb + (nb % 2)


EG = E + N
NBG = _even_blocks(EG, BG)
NBSA = _even_blocks(E, BG)
NBC = _even_blocks(E, BC)
EGP = NSUB * NBG * BG
ESPA = NSUB * NBSA * BG
ESPC = NSUB * NBC * BC

RB = 512
GRID = NP // RB


def _dot(a, b):
    return jnp.dot(a, b, preferred_element_type=F32)


def _tc_pre_body(x_ref, wg_ref, asrc_ref, adst_ref,
                 xwp_ref, xp_ref, adt_ref):
    x = x_ref[...]
    xw = _dot(x, wg_ref[...])
    as_ = _dot(xw, asrc_ref[...])
    ad_ = _dot(xw, adst_ref[...])
    ones = jnp.ones((RB, 1), F32)
    xwp_ref[...] = jnp.concatenate(
        [xw, ones, as_, jnp.zeros((RB, DW - DI - 2), F32)], axis=1)
    xp_ref[...] = jnp.concatenate(
        [x, ones, jnp.zeros((RB, DW - DI - 1), F32)], axis=1)
    adt_ref[...] = jnp.concatenate([ad_, jnp.zeros((RB, 15), F32)], axis=1)


def _tc_pre(xP, W_gat, a_src, a_dst):
    row = lambda i: (i, 0)
    full = lambda i: (0, 0)
    return pl.pallas_call(
        _tc_pre_body,
        grid=(GRID,),
        in_specs=[
            pl.BlockSpec((RB, DI), row),
            pl.BlockSpec((DI, H2), full),
            pl.BlockSpec((H2, 1), full),
            pl.BlockSpec((H2, 1), full),
        ],
        out_specs=[
            pl.BlockSpec((RB, DW), row),
            pl.BlockSpec((RB, DW), row),
            pl.BlockSpec((RB, 16), row),
        ],
        out_shape=[
            jax.ShapeDtypeStruct((NP, DW), F32),
            jax.ShapeDtypeStruct((NP, DW), F32),
            jax.ShapeDtypeStruct((NP, 16), F32),
        ],
    )(xP, W_gat, a_src, a_dst)


def _layernorm(h, g, b):
    mu = jnp.mean(h, axis=1, keepdims=True)
    d = h - mu
    var = jnp.mean(d * d, axis=1, keepdims=True)
    return d * lax.rsqrt(var + 1e-5) * g + b


def _tc_mid_body(md_ref, ag_ref, x_ref, bgat_ref, w1l_ref, b1l_ref,
                 w1r_ref, g1_ref, be1_ref, w2r_ref,
                 hlo_ref, hhi_ref, hr_ref):
    md = md_ref[...]
    ag = ag_ref[...]
    x1 = md[:, :H2] / jnp.maximum(md[:, H2:H2 + 1], 1e-16) + bgat_ref[...]
    mean1 = ag[:, :DI] / jnp.maximum(ag[:, DI:DI + 1], 1.0)
    x2 = _dot(mean1, w1l_ref[...]) + b1l_ref[...] + _dot(x_ref[...], w1r_ref[...])
    h = jnp.concatenate([x1, x2], axis=1)
    h = _layernorm(h, g1_ref[...], be1_ref[...])
    h = jnp.maximum(h, 0.0)
    hlo_ref[...] = h[:, :H2]
    hhi_ref[...] = h[:, H2:]
    hr_ref[...] = _dot(h, w2r_ref[...])


def _tc_mid(md, ag, xP, bgat, W1_l, b1l, W1_r, g1, be1, W2_r):
    row = lambda i: (i, 0)
    full = lambda i: (0, 0)
    return pl.pallas_call(
        _tc_mid_body,
        grid=(GRID,),
        in_specs=[
            pl.BlockSpec((RB, DW), row),
            pl.BlockSpec((RB, DW), row),
            pl.BlockSpec((RB, DI), row),
            pl.BlockSpec((1, H2), full),
            pl.BlockSpec((DI, H2), full),
            pl.BlockSpec((1, H2), full),
            pl.BlockSpec((DI, H2), full),
            pl.BlockSpec((1, HID), full),
            pl.BlockSpec((1, HID), full),
            pl.BlockSpec((HID, HID), full),
        ],
        out_specs=[
            pl.BlockSpec((RB, H2), row),
            pl.BlockSpec((RB, H2), row),
            pl.BlockSpec((RB, HID), row),
        ],
        out_shape=[
            jax.ShapeDtypeStruct((NP, H2), F32),
            jax.ShapeDtypeStruct((NP, H2), F32),
            jax.ShapeDtypeStruct((NP, HID), F32),
        ],
    )(md, ag, xP, bgat, W1_l, b1l, W1_r, g1, be1, W2_r)


def _tc_fin_body(alo_ref, ahi_ref, deg_ref, hr_ref, w2l_ref, b2l_ref,
                 g2_ref, be2_ref, wc_ref, bc_ref, out_ref):
    deg = jnp.maximum(deg_ref[...], 1.0)
    m2l = alo_ref[...] / deg
    m2h = ahi_ref[...] / deg
    w2l = w2l_ref[...]
    h2 = (_dot(m2l, w2l[:H2, :]) + _dot(m2h, w2l[H2:, :])
          + b2l_ref[...] + hr_ref[...])
    h2 = _layernorm(h2, g2_ref[...], be2_ref[...])
    h2 = jnp.maximum(h2, 0.0)
    out_ref[...] = _dot(h2, wc_ref[...]) + bc_ref[...]


def _tc_fin(alo, ahi, deg, hr, W2_l, b2l, g2, be2, Wc, bc):
    row = lambda i: (i, 0)
    full = lambda i: (0, 0)
    return pl.pallas_call(
        _tc_fin_body,
        grid=(GRID,),
        in_specs=[
            pl.BlockSpec((RB, H2), row),
            pl.BlockSpec((RB, H2), row),
            pl.BlockSpec((RB, 1), row),
            pl.BlockSpec((RB, HID), row),
            pl.BlockSpec((HID, HID), full),
            pl.BlockSpec((1, HID), full),
            pl.BlockSpec((1, HID), full),
            pl.BlockSpec((1, HID), full),
            pl.BlockSpec((HID, 1), full),
            pl.BlockSpec((1, 1), full),
        ],
        out_specs=pl.BlockSpec((RB, 1), row),
        out_shape=jax.ShapeDtypeStruct((NP, 1), F32),
    )(alo, ahi, deg, hr, W2_l, b2l, g2, be2, Wc, bc)


def _zero_acc(r0, acc, width, sid):
    zv = jnp.zeros((16,), F32)
    for r in range(16):
        for k in range(width // 16):
            r0[r, pl.ds(k * 16, 16)] = zv
    base = sid * RPT

    def body(i, _):
        off = pl.multiple_of(base + i * 16, 16)
        pltpu.sync_copy(r0.at[pl.ds(0, 16)], acc.at[pl.ds(off, 16)])
        return 0

    lax.fori_loop(0, RPT // 16, body, 0)


def _pipe_plain(nblk, sid, src2, dst2, table, bufs, acc):
    (s0, d0, r0, g0, ss0), (s1, d1, r1, g1, ss1) = bufs
    half = nblk // 2

    def fire(bid, sidx, didx, gsem, rows):
        pltpu.sync_copy(src2.at[bid], sidx)
        pltpu.sync_copy(dst2.at[bid], didx)
        pltpu.async_copy(table.at[sidx], rows, gsem)

    fire(sid * nblk, s0, d0, g0, r0)
    fire(sid * nblk + 1, s1, d1, g1, r1)

    def body(i, _):
        b0 = sid * nblk + 2 * i
        pltpu.make_async_copy(table.at[s0], r0, g0).wait()
        pltpu.async_copy(r0, acc.at[d0], ss0, add=True)
        pltpu.make_async_copy(table.at[s1], r1, g1).wait()
        pltpu.async_copy(r1, acc.at[d1], ss1, add=True)

        @pl.when(i < half - 1)
        def _():
            pltpu.make_async_copy(r0, acc.at[d0], ss0).wait()
            fire(b0 + 2, s0, d0, g0, r0)
            pltpu.make_async_copy(r1, acc.at[d1], ss1).wait()
            fire(b0 + 3, s1, d1, g1, r1)

        return 0

    lax.fori_loop(0, half, body, 0)
    pltpu.make_async_copy(r0, acc.at[d0], ss0).wait()
    pltpu.make_async_copy(r1, acc.at[d1], ss1).wait()


def _pipe_gat(sid, src2, dst2, xwp, adt, bufs, acc):
    (s0, d0, r0, a0, g0, ss0), (s1, d1, r1, a1, g1, ss1) = bufs
    half = NBG // 2
    lane = lax.iota(jnp.int32, 16)
    as_col = jnp.full((16,), DI + 1, jnp.int32)
    zero_col = jnp.zeros((16,), jnp.int32)

    def fire(bid, sidx, didx, gsem, rows, adr):
        pltpu.sync_copy(src2.at[bid], sidx)
        pltpu.sync_copy(dst2.at[bid], didx)
        pltpu.async_copy(xwp.at[sidx], rows, gsem)
        pltpu.async_copy(adt.at[didx], adr, gsem)

    def process(sidx, didx, gsem, rows, adr, ssem):
        pltpu.make_async_copy(xwp.at[sidx], rows, gsem).wait()
        pltpu.make_async_copy(adt.at[didx], adr, gsem).wait()
        for j8 in range(BG // 16):
            jvec = lane + (j8 * 16)
            u = (plsc.load_gather(rows, [jvec, as_col])
                 + plsc.load_gather(adr, [jvec, zero_col]))
            wv = jnp.exp(jnp.maximum(u, 0.2 * u))
            for l in range(16):
                w = wv[l]
                j = j8 * 16 + l
                for k in range(DI // 16):
                    ksl = pl.ds(k * 16, 16)
                    rows[j, ksl] = rows[j, ksl] * w
                rows[j, pl.ds(DI, 16)] = lax.broadcast(w, (16,))
        pltpu.async_copy(rows, acc.at[didx], ssem, add=True)

    fire(sid * NBG, s0, d0, g0, r0, a0)
    fire(sid * NBG + 1, s1, d1, g1, r1, a1)

    def body(i, _):
        b0 = sid * NBG + 2 * i
        process(s0, d0, g0, r0, a0, ss0)
        process(s1, d1, g1, r1, a1, ss1)

        @pl.when(i < half - 1)
        def _():
            pltpu.make_async_copy(r0, acc.at[d0], ss0).wait()
            fire(b0 + 2, s0, d0, g0, r0, a0)
            pltpu.make_async_copy(r1, acc.at[d1], ss1).wait()
            fire(b0 + 3, s1, d1, g1, r1, a1)

        return 0

    lax.fori_loop(0, half, body, 0)
    pltpu.make_async_copy(r0, acc.at[d0], ss0).wait()
    pltpu.make_async_copy(r1, acc.at[d1], ss1).wait()


def _sc_stage_a_body(xwp, xp, adt, gsrc2, gdst2, ssrc2, sdst2,
                     md_out, ag_out,
                     s0, s1, d0, d1, r0, r1, a0, a1,
                     g0, g1, ss0, ss1, acc):
    cid = lax.axis_index("c")
    sid = lax.axis_index("s")

    _zero_acc(r0, acc, DW, sid)
    plsc.subcore_barrier()

    @pl.when(cid == 0)
    def _():
        _pipe_gat(sid, gsrc2, gdst2, xwp, adt,
                  ((s0, d0, r0, a0, g0, ss0), (s1, d1, r1, a1, g1, ss1)),
                  acc)

    @pl.when(cid == 1)
    def _():
        _pipe_plain(NBSA, sid, ssrc2, sdst2, xp,
                    ((s0, d0, r0, g0, ss0), (s1, d1, r1, g1, ss1)), acc)

    plsc.subcore_barrier()

    base = pl.multiple_of(sid * RPT, 16)

    @pl.when(cid == 0)
    def _():
        pltpu.sync_copy(acc.at[pl.ds(base, RPT)], md_out.at[pl.ds(base, RPT)])

    @pl.when(cid == 1)
    def _():
        pltpu.sync_copy(acc.at[pl.ds(base, RPT)], ag_out.at[pl.ds(base, RPT)])


def _sc_stage_a(xwp, xp, adt, gsrc2, gdst2, ssrc2, sdst2):
    mesh = plsc.VectorSubcoreMesh(core_axis_name="c", subcore_axis_name="s")
    return pl.kernel(
        _sc_stage_a_body,
        out_type=[
            jax.ShapeDtypeStruct((NP, DW), F32),
            jax.ShapeDtypeStruct((NP, DW), F32),
        ],
        mesh=mesh,
        compiler_params=pltpu.CompilerParams(
            needs_layout_passes=False, use_tc_tiling_on_sc=False),
        scratch_types=[
            pltpu.VMEM((BG,), jnp.int32),
            pltpu.VMEM((BG,), jnp.int32),
            pltpu.VMEM((BG,), jnp.int32),
            pltpu.VMEM((BG,), jnp.int32),
            pltpu.VMEM((BG, DW), F32),
            pltpu.VMEM((BG, DW), F32),
            pltpu.VMEM((BG, 16), F32),
            pltpu.VMEM((BG, 16), F32),
            pltpu.SemaphoreType.DMA,
            pltpu.SemaphoreType.DMA,
            pltpu.SemaphoreType.DMA,
            pltpu.SemaphoreType.DMA,
            pltpu.VMEM_SHARED((NP, DW), F32),
        ],
    )(xwp, xp, adt, gsrc2, gdst2, ssrc2, sdst2)


def _sc_stage_c_body(hlo, hhi, ssrc2, sdst2, alo_out, ahi_out,
                     s0, s1, d0, d1, r0, r1, g0, g1, ss0, ss1, acc):
    cid = lax.axis_index("c")
    sid = lax.axis_index("s")

    _zero_acc(r0, acc, H2, sid)
    plsc.subcore_barrier()

    bufs = ((s0, d0, r0, g0, ss0), (s1, d1, r1, g1, ss1))

    @pl.when(cid == 0)
    def _():
        _pipe_plain(NBC, sid, ssrc2, sdst2, hlo, bufs, acc)

    @pl.when(cid == 1)
    def _():
        _pipe_plain(NBC, sid, ssrc2, sdst2, hhi, bufs, acc)

    plsc.subcore_barrier()

    base = pl.multiple_of(sid * RPT, 16)

    @pl.when(cid == 0)
    def _():
        pltpu.sync_copy(acc.at[pl.ds(base, RPT)], alo_out.at[pl.ds(base, RPT)])

    @pl.when(cid == 1)
    def _():
        pltpu.sync_copy(acc.at[pl.ds(base, RPT)], ahi_out.at[pl.ds(base, RPT)])


def _sc_stage_c(hlo, hhi, ssrc2, sdst2):
    mesh = plsc.VectorSubcoreMesh(core_axis_name="c", subcore_axis_name="s")
    return pl.kernel(
        _sc_stage_c_body,
        out_type=[
            jax.ShapeDtypeStruct((NP, H2), F32),
            jax.ShapeDtypeStruct((NP, H2), F32),
        ],
        mesh=mesh,
        compiler_params=pltpu.CompilerParams(
            needs_layout_passes=False, use_tc_tiling_on_sc=False),
        scratch_types=[
            pltpu.VMEM((BC,), jnp.int32),
            pltpu.VMEM((BC,), jnp.int32),
            pltpu.VMEM((BC,), jnp.int32),
            pltpu.VMEM((BC,), jnp.int32),
            pltpu.VMEM((BC, H2), F32),
            pltpu.VMEM((BC, H2), F32),
            pltpu.SemaphoreType.DMA,
            pltpu.SemaphoreType.DMA,
            pltpu.SemaphoreType.DMA,
            pltpu.SemaphoreType.DMA,
            pltpu.VMEM_SHARED((NP, H2), F32),
        ],
    )(hlo, hhi, ssrc2, sdst2)


def kernel(x, edge_index, W_gat, att_src, att_dst, b_gat, W1_l, b1_l, W1_r,
           g1, be1, W2_l, b2_l, W2_r, g2, be2, Wc, bc):
    src = edge_index[0]
    dst = edge_index[1]
    loops = jnp.arange(N, dtype=jnp.int32)

    gsrc = jnp.concatenate([src, loops])
    gdst = jnp.concatenate([dst, loops])
    gsrc2 = jnp.pad(gsrc, (0, EGP - EG)).reshape(NSUB * NBG, BG)
    gdst2 = jnp.pad(gdst, (0, EGP - EG), constant_values=N).reshape(NSUB * NBG, BG)
    ssrcA = jnp.pad(src, (0, ESPA - E)).reshape(NSUB * NBSA, BG)
    sdstA = jnp.pad(dst, (0, ESPA - E), constant_values=N).reshape(NSUB * NBSA, BG)
    ssrcC = jnp.pad(src, (0, ESPC - E)).reshape(NSUB * NBC, BC)
    sdstC = jnp.pad(dst, (0, ESPC - E), constant_values=N).reshape(NSUB * NBC, BC)

    xP = jnp.pad(x, ((0, NP - N), (0, 0)))

    xwp, xp, adt = _tc_pre(xP, W_gat,
                           att_src.reshape(H2, 1), att_dst.reshape(H2, 1))

    md, ag = _sc_stage_a(xwp, xp, adt, gsrc2, gdst2, ssrcA, sdstA)

    hlo, hhi, hr = _tc_mid(md, ag, xP, b_gat.reshape(1, H2), W1_l,
                           b1_l.reshape(1, H2), W1_r, g1.reshape(1, HID),
                           be1.reshape(1, HID), W2_r)

    alo, ahi = _sc_stage_c(hlo, hhi, ssrcC, sdstC)

    deg = lax.slice(ag, (0, DI), (NP, DI + 1))
    out = _tc_fin(alo, ahi, deg, hr, W2_l, b2_l.reshape(1, HID),
                  g2.reshape(1, HID), be2.reshape(1, HID), Wc,
                  bc.reshape(1, 1))
    return out[:N, 0]
